# speedup vs baseline: 1.3002x; 1.3002x over previous
import functools

import jax
import jax.numpy as jnp
from jax.experimental import pallas as pl


def _body(mean_ref, scale_ref, noise_ref, winw_ref, winb_ref, woutw_ref,
          woutb_ref, cb_ref, out_ref, *, n_codebooks, cb_size, tblk):
    mean = mean_ref[0]
    scale = scale_ref[0]
    noise = noise_ref[0]

    stdev = jax.nn.softplus(scale) + 1e-4
    z = noise * stdev + mean
    res = z
    acc = jnp.zeros_like(z)

    iota = jax.lax.broadcasted_iota(jnp.int32, (cb_size, tblk), 0)

    for i in range(n_codebooks):
        w_in = winw_ref[i]
        enc = jax.lax.dot_general(
            w_in, res, (((1,), (0,)), ((), ()))) + winb_ref[i]
        ssq = jnp.sum(enc * enc, axis=0, keepdims=True)
        encn = enc / jnp.maximum(jnp.sqrt(ssq), 1e-12)

        cb = cb_ref[i]
        cbssq = jnp.sum(cb * cb, axis=1, keepdims=True)
        cbn = cb / jnp.maximum(jnp.sqrt(cbssq), 1e-12)

        u = jax.lax.dot_general(
            cbn, encn, (((1,), (0,)), ((), ())))
        encsq = jnp.sum(encn * encn, axis=0, keepdims=True)
        cbnsq = jnp.sum(cbn * cbn, axis=1, keepdims=True)
        dist = encsq - 2.0 * u + cbnsq

        m = jnp.min(dist, axis=0, keepdims=True)
        cand = jnp.where(dist == m, iota, cb_size)
        idx = jnp.min(cand, axis=0, keepdims=True)
        oh = (iota == idx).astype(jnp.float32)

        zq = jax.lax.dot_general(
            cb, oh, (((0,), (0,)), ((), ())),
            precision=jax.lax.Precision.HIGHEST)
        zq_st = enc + (zq - enc)

        w_out = woutw_ref[i]
        out = jax.lax.dot_general(
            w_out, zq_st, (((1,), (0,)), ((), ()))) + woutb_ref[i]
        acc = acc + out
        res = res - out

    out_ref[0] = acc


def kernel(x, noise, in_proj_w, in_proj_b, out_proj_w, out_proj_b, codebooks):
    bsz, c2, t = x.shape
    d = c2 // 2
    ncb, cb_size, cd = codebooks.shape
    tblk = 512 if t % 512 == 0 else t
    grid = (bsz, t // tblk)

    in_b = in_proj_b.reshape(ncb, cd, 1)
    out_b = out_proj_b.reshape(ncb, d, 1)

    body = functools.partial(_body, n_codebooks=ncb, cb_size=cb_size,
                             tblk=tblk)

    tok_spec = pl.BlockSpec((1, d, tblk), lambda b, tt: (b, 0, tt))
    full = lambda shape: pl.BlockSpec(shape, lambda b, tt: (0,) * len(shape))

    return pl.pallas_call(
        body,
        grid=grid,
        in_specs=[
            tok_spec,
            pl.BlockSpec((1, d, tblk), lambda b, tt: (b, 1, tt)),
            tok_spec,
            full((ncb, cd, d)),
            full((ncb, cd, 1)),
            full((ncb, d, cd)),
            full((ncb, d, 1)),
            full((ncb, cb_size, cd)),
        ],
        out_specs=tok_spec,
        out_shape=jax.ShapeDtypeStruct((bsz, d, t), jnp.float32),
    )(x, x, noise, in_proj_w, in_b, out_proj_w, out_b, codebooks)


# speedup vs baseline: 1.9136x; 1.4719x over previous
import functools

import jax
import jax.numpy as jnp
from jax.experimental import pallas as pl


def _body(mean_ref, scale_ref, noise_ref, winw_ref, winb_ref, woutw_ref,
          woutb_ref, cb_ref, cbt_ref, out_ref, *, n_codebooks, cb_size, tblk):
    mean = mean_ref[0]
    scale = scale_ref[0]
    noise = noise_ref[0]

    stdev = jax.nn.softplus(scale) + 1e-4
    z = noise * stdev + mean
    res = z
    acc = jnp.zeros_like(z)

    iota = jax.lax.broadcasted_iota(jnp.int32, (cb_size, tblk), 0)

    for i in range(n_codebooks):
        w_in = winw_ref[i]
        enc = jax.lax.dot_general(
            w_in, res, (((1,), (0,)), ((), ()))) + winb_ref[i]
        ssq = jnp.sum(enc * enc, axis=0, keepdims=True)
        encn = enc / jnp.maximum(jnp.sqrt(ssq), 1e-12)

        cb = cb_ref[i]
        cbssq = jnp.sum(cb * cb, axis=1, keepdims=True)
        cbn = cb / jnp.maximum(jnp.sqrt(cbssq), 1e-12)

        u = jax.lax.dot_general(
            cbn, encn, (((1,), (0,)), ((), ())))
        encsq = jnp.sum(encn * encn, axis=0, keepdims=True)
        cbnsq = jnp.sum(cbn * cbn, axis=1, keepdims=True)
        dist = encsq - 2.0 * u + cbnsq

        m = jnp.min(dist, axis=0, keepdims=True)
        cand = jnp.where(dist == m, iota, cb_size)
        idx = jnp.min(cand, axis=0, keepdims=True)
        oh = (iota == idx).astype(jnp.bfloat16)

        cbt = cbt_ref[i]
        hi = cbt.astype(jnp.bfloat16)
        r1 = cbt - hi.astype(jnp.float32)
        mid = r1.astype(jnp.bfloat16)
        lo = (r1 - mid.astype(jnp.float32)).astype(jnp.bfloat16)
        dn = (((1,), (0,)), ((), ()))
        f32 = jnp.float32
        zq = (jax.lax.dot_general(hi, oh, dn, preferred_element_type=f32)
              + jax.lax.dot_general(mid, oh, dn, preferred_element_type=f32)
              + jax.lax.dot_general(lo, oh, dn, preferred_element_type=f32))
        zq_st = enc + (zq - enc)

        w_out = woutw_ref[i]
        out = jax.lax.dot_general(
            w_out, zq_st, (((1,), (0,)), ((), ()))) + woutb_ref[i]
        acc = acc + out
        res = res - out

    out_ref[0] = acc


def kernel(x, noise, in_proj_w, in_proj_b, out_proj_w, out_proj_b, codebooks):
    bsz, c2, t = x.shape
    d = c2 // 2
    ncb, cb_size, cd = codebooks.shape
    tblk = 512 if t % 512 == 0 else t
    grid = (bsz, t // tblk)

    in_b = in_proj_b.reshape(ncb, cd, 1)
    out_b = out_proj_b.reshape(ncb, d, 1)
    cbt = jnp.transpose(codebooks, (0, 2, 1))

    body = functools.partial(_body, n_codebooks=ncb, cb_size=cb_size,
                             tblk=tblk)

    tok_spec = pl.BlockSpec((1, d, tblk), lambda b, tt: (b, 0, tt))
    full = lambda shape: pl.BlockSpec(shape, lambda b, tt: (0,) * len(shape))

    return pl.pallas_call(
        body,
        grid=grid,
        in_specs=[
            tok_spec,
            pl.BlockSpec((1, d, tblk), lambda b, tt: (b, 1, tt)),
            tok_spec,
            full((ncb, cd, d)),
            full((ncb, cd, 1)),
            full((ncb, d, cd)),
            full((ncb, d, 1)),
            full((ncb, cb_size, cd)),
            full((ncb, cd, cb_size)),
        ],
        out_specs=tok_spec,
        out_shape=jax.ShapeDtypeStruct((bsz, d, t), jnp.float32),
    )(x, x, noise, in_proj_w, in_b, out_proj_w, out_b, codebooks, cbt)


# speedup vs baseline: 2.5454x; 1.3301x over previous
import functools

import jax
import jax.numpy as jnp
from jax.experimental import pallas as pl


def _prep_body(cb_ref, cbt_ref, cbna_ref, hi_ref, mid_ref, lo_ref):
    cb = cb_ref[...]
    cbssq = jnp.sum(cb * cb, axis=2, keepdims=True)
    cbn = cb / jnp.maximum(jnp.sqrt(cbssq), 1e-12)
    chalf = jnp.sum(cbn * cbn, axis=2, keepdims=True) * 0.5
    cbna_ref[...] = jnp.concatenate([cbn, -chalf], axis=2)

    cbt = cbt_ref[...]
    hi = cbt.astype(jnp.bfloat16)
    r1 = cbt - hi.astype(jnp.float32)
    mid = r1.astype(jnp.bfloat16)
    lo = (r1 - mid.astype(jnp.float32)).astype(jnp.bfloat16)
    hi_ref[...] = hi
    mid_ref[...] = mid
    lo_ref[...] = lo


def _body(mean_ref, scale_ref, noise_ref, winw_ref, winb_ref, woutw_ref,
          woutb_ref, cbna_ref, hi_ref, mid_ref, lo_ref, out_ref,
          *, n_codebooks, tblk):
    mean = mean_ref[0]
    scale = scale_ref[0]
    noise = noise_ref[0]

    stdev = jax.nn.softplus(scale) + 1e-4
    z = noise * stdev + mean
    res = z
    acc = jnp.zeros_like(z)

    ones = jnp.ones((1, tblk), jnp.float32)
    dn = (((1,), (0,)), ((), ()))
    f32 = jnp.float32

    for i in range(n_codebooks):
        w_in = winw_ref[i]
        enc = jax.lax.dot_general(w_in, res, dn) + winb_ref[i]
        ssq = jnp.sum(enc * enc, axis=0, keepdims=True)
        encn = enc / jnp.maximum(jnp.sqrt(ssq), 1e-12)
        encn_aug = jnp.concatenate([encn, ones], axis=0)

        s = jax.lax.dot_general(cbna_ref[i], encn_aug, dn)
        m = jnp.max(s, axis=0, keepdims=True)
        oh = (s == m).astype(jnp.bfloat16)

        zq = (jax.lax.dot_general(hi_ref[i], oh, dn, preferred_element_type=f32)
              + jax.lax.dot_general(mid_ref[i], oh, dn, preferred_element_type=f32)
              + jax.lax.dot_general(lo_ref[i], oh, dn, preferred_element_type=f32))
        zq_st = enc + (zq - enc)

        out = jax.lax.dot_general(woutw_ref[i], zq_st, dn) + woutb_ref[i]
        acc = acc + out
        res = res - out

    out_ref[0] = acc


def kernel(x, noise, in_proj_w, in_proj_b, out_proj_w, out_proj_b, codebooks):
    bsz, c2, t = x.shape
    d = c2 // 2
    ncb, cb_size, cd = codebooks.shape
    tblk = 512 if t % 512 == 0 else t
    grid = (bsz, t // tblk)

    in_b = in_proj_b.reshape(ncb, cd, 1)
    out_b = out_proj_b.reshape(ncb, d, 1)
    cbt = jnp.transpose(codebooks, (0, 2, 1))

    bf16 = jnp.bfloat16
    cbna, hi, mid, lo = pl.pallas_call(
        _prep_body,
        out_shape=[
            jax.ShapeDtypeStruct((ncb, cb_size, cd + 1), jnp.float32),
            jax.ShapeDtypeStruct((ncb, cd, cb_size), bf16),
            jax.ShapeDtypeStruct((ncb, cd, cb_size), bf16),
            jax.ShapeDtypeStruct((ncb, cd, cb_size), bf16),
        ],
    )(codebooks, cbt)

    body = functools.partial(_body, n_codebooks=ncb, tblk=tblk)

    tok_spec = pl.BlockSpec((1, d, tblk), lambda b, tt: (b, 0, tt))
    full = lambda shape: pl.BlockSpec(shape, lambda b, tt: (0,) * len(shape))

    return pl.pallas_call(
        body,
        grid=grid,
        in_specs=[
            tok_spec,
            pl.BlockSpec((1, d, tblk), lambda b, tt: (b, 1, tt)),
            tok_spec,
            full((ncb, cd, d)),
            full((ncb, cd, 1)),
            full((ncb, d, cd)),
            full((ncb, d, 1)),
            full((ncb, cb_size, cd + 1)),
            full((ncb, cd, cb_size)),
            full((ncb, cd, cb_size)),
            full((ncb, cd, cb_size)),
        ],
        out_specs=tok_spec,
        out_shape=jax.ShapeDtypeStruct((bsz, d, t), jnp.float32),
    )(x, x, noise, in_proj_w, in_b, out_proj_w, out_b, cbna, hi, mid, lo)


# speedup vs baseline: 3.1551x; 1.2395x over previous
import functools

import jax
import jax.numpy as jnp
from jax.experimental import pallas as pl


def _prep_body(cb_ref, cbt_ref, woutw_ref, cbna_ref, q_ref, *, n_codebooks):
    cb = cb_ref[...]
    cbssq = jnp.sum(cb * cb, axis=2, keepdims=True)
    cbn = cb / jnp.maximum(jnp.sqrt(cbssq), 1e-12)
    chalf = jnp.sum(cbn * cbn, axis=2, keepdims=True) * 0.5
    cbna_ref[...] = jnp.concatenate([cbn, -chalf], axis=2)

    dn = (((1,), (0,)), ((), ()))
    for i in range(n_codebooks):
        q = jax.lax.dot_general(woutw_ref[i], cbt_ref[i], dn)
        hi = q.astype(jnp.bfloat16)
        r1 = q - hi.astype(jnp.float32)
        mid = r1.astype(jnp.bfloat16)
        lo = (r1 - mid.astype(jnp.float32)).astype(jnp.bfloat16)
        q_ref[i] = jnp.concatenate([hi, mid, lo], axis=0)


def _body(mean_ref, scale_ref, noise_ref, winw_ref, winb_ref, woutb_ref,
          cbna_ref, q_ref, out_ref, *, n_codebooks, tblk, d):
    mean = mean_ref[0]
    scale = scale_ref[0]
    noise = noise_ref[0]

    stdev = jax.nn.softplus(scale) + 1e-4
    z = noise * stdev + mean
    res = z
    acc = jnp.zeros_like(z)

    ones = jnp.ones((1, tblk), jnp.float32)
    dn = (((1,), (0,)), ((), ()))
    f32 = jnp.float32

    for i in range(n_codebooks):
        enc = jax.lax.dot_general(winw_ref[i], res, dn) + winb_ref[i]
        ssq = jnp.sum(enc * enc, axis=0, keepdims=True)
        encn = enc / jnp.maximum(jnp.sqrt(ssq), 1e-12)
        encn_aug = jnp.concatenate([encn, ones], axis=0)

        s = jax.lax.dot_general(cbna_ref[i], encn_aug, dn)
        m = jnp.max(s, axis=0, keepdims=True)
        oh = (s == m).astype(jnp.bfloat16)

        g = jax.lax.dot_general(q_ref[i], oh, dn,
                                preferred_element_type=f32)
        out = ((g[0:d] + g[d:2 * d]) + g[2 * d:3 * d]) + woutb_ref[i]
        acc = acc + out
        res = res - out

    out_ref[0] = acc


def kernel(x, noise, in_proj_w, in_proj_b, out_proj_w, out_proj_b, codebooks):
    bsz, c2, t = x.shape
    d = c2 // 2
    ncb, cb_size, cd = codebooks.shape
    tblk = 512 if t % 512 == 0 else t
    grid = (bsz, t // tblk)

    in_b = in_proj_b.reshape(ncb, cd, 1)
    out_b = out_proj_b.reshape(ncb, d, 1)
    cbt = jnp.transpose(codebooks, (0, 2, 1))

    cbna, q = pl.pallas_call(
        functools.partial(_prep_body, n_codebooks=ncb),
        out_shape=[
            jax.ShapeDtypeStruct((ncb, cb_size, cd + 1), jnp.float32),
            jax.ShapeDtypeStruct((ncb, 3 * d, cb_size), jnp.bfloat16),
        ],
    )(codebooks, cbt, out_proj_w)

    body = functools.partial(_body, n_codebooks=ncb, tblk=tblk, d=d)

    tok_spec = pl.BlockSpec((1, d, tblk), lambda b, tt: (b, 0, tt))
    full = lambda shape: pl.BlockSpec(shape, lambda b, tt: (0,) * len(shape))

    return pl.pallas_call(
        body,
        grid=grid,
        in_specs=[
            tok_spec,
            pl.BlockSpec((1, d, tblk), lambda b, tt: (b, 1, tt)),
            tok_spec,
            full((ncb, cd, d)),
            full((ncb, cd, 1)),
            full((ncb, d, 1)),
            full((ncb, cb_size, cd + 1)),
            full((ncb, 3 * d, cb_size)),
        ],
        out_specs=tok_spec,
        out_shape=jax.ShapeDtypeStruct((bsz, d, t), jnp.float32),
    )(x, x, noise, in_proj_w, in_b, out_b, cbna, q)


# speedup vs baseline: 3.5859x; 1.1365x over previous
import functools

import jax
import jax.numpy as jnp
from jax.experimental import pallas as pl


def _prep_body(cb_ref, cbt_ref, woutw_ref, cbna_ref, q_ref, *, n_codebooks):
    cb = cb_ref[...]
    cbssq = jnp.sum(cb * cb, axis=2, keepdims=True)
    cbn = cb / jnp.maximum(jnp.sqrt(cbssq), 1e-12)
    chalf = jnp.sum(cbn * cbn, axis=2, keepdims=True) * 0.5
    cbna_ref[...] = jnp.concatenate([cbn, -chalf], axis=2)

    dn = (((1,), (0,)), ((), ()))
    for i in range(n_codebooks):
        q = jax.lax.dot_general(woutw_ref[i], cbt_ref[i], dn)
        hi = q.astype(jnp.bfloat16)
        r1 = q - hi.astype(jnp.float32)
        mid = r1.astype(jnp.bfloat16)
        lo = (r1 - mid.astype(jnp.float32)).astype(jnp.bfloat16)
        q_ref[i] = jnp.concatenate([hi, mid, lo], axis=0)


def _body(mean_ref, scale_ref, noise_ref, winw_ref, winb_ref, woutb_ref,
          cbna_ref, q_ref, out_ref, *, n_codebooks, tblk, d, nchain):
    mean = mean_ref[0]
    scale = scale_ref[0]
    noise = noise_ref[0]

    stdev = jax.nn.softplus(scale) + 1e-4
    z = noise * stdev + mean

    cw = tblk // nchain
    ones = jnp.ones((1, cw), jnp.float32)
    dn = (((1,), (0,)), ((), ()))
    f32 = jnp.float32

    res = [z[:, h * cw:(h + 1) * cw] for h in range(nchain)]
    acc = [jnp.zeros((d, cw), f32) for _ in range(nchain)]

    for i in range(n_codebooks):
        for h in range(nchain):
            enc = jax.lax.dot_general(winw_ref[i], res[h], dn) + winb_ref[i]
            ssq = jnp.sum(enc * enc, axis=0, keepdims=True)
            encn = enc / jnp.maximum(jnp.sqrt(ssq), 1e-12)
            encn_aug = jnp.concatenate([encn, ones], axis=0)

            s = jax.lax.dot_general(cbna_ref[i], encn_aug, dn)
            m = jnp.max(s, axis=0, keepdims=True)
            oh = (s == m).astype(jnp.bfloat16)

            g = jax.lax.dot_general(q_ref[i], oh, dn,
                                    preferred_element_type=f32)
            out = ((g[0:d] + g[d:2 * d]) + g[2 * d:3 * d]) + woutb_ref[i]
            acc[h] = acc[h] + out
            res[h] = res[h] - out

    out_ref[0] = jnp.concatenate(acc, axis=1)


def kernel(x, noise, in_proj_w, in_proj_b, out_proj_w, out_proj_b, codebooks):
    bsz, c2, t = x.shape
    d = c2 // 2
    ncb, cb_size, cd = codebooks.shape
    tblk = 1024 if t % 1024 == 0 else t
    nchain = 2 if tblk % 1024 == 0 else 1
    grid = (bsz, t // tblk)

    in_b = in_proj_b.reshape(ncb, cd, 1)
    out_b = out_proj_b.reshape(ncb, d, 1)
    cbt = jnp.transpose(codebooks, (0, 2, 1))

    cbna, q = pl.pallas_call(
        functools.partial(_prep_body, n_codebooks=ncb),
        out_shape=[
            jax.ShapeDtypeStruct((ncb, cb_size, cd + 1), jnp.float32),
            jax.ShapeDtypeStruct((ncb, 3 * d, cb_size), jnp.bfloat16),
        ],
    )(codebooks, cbt, out_proj_w)

    body = functools.partial(_body, n_codebooks=ncb, tblk=tblk, d=d,
                             nchain=nchain)

    tok_spec = pl.BlockSpec((1, d, tblk), lambda b, tt: (b, 0, tt))
    full = lambda shape: pl.BlockSpec(shape, lambda b, tt: (0,) * len(shape))

    return pl.pallas_call(
        body,
        grid=grid,
        in_specs=[
            tok_spec,
            pl.BlockSpec((1, d, tblk), lambda b, tt: (b, 1, tt)),
            tok_spec,
            full((ncb, cd, d)),
            full((ncb, cd, 1)),
            full((ncb, d, 1)),
            full((ncb, cb_size, cd + 1)),
            full((ncb, 3 * d, cb_size)),
        ],
        out_specs=tok_spec,
        out_shape=jax.ShapeDtypeStruct((bsz, d, t), jnp.float32),
    )(x, x, noise, in_proj_w, in_b, out_b, cbna, q)
